# Initial kernel scaffold; baseline (speedup 1.0000x reference)
import jax
import jax.numpy as jnp
from jax.experimental import pallas as pl


def kernel(nodes, ur_history_lists, rating_history_lists, u2e_w, r2e_w, rating2e_w, w_r1_w, w_r1_b, w_r2_w, w_r2_b, att1_w, att1_b, att2_w, att2_b, att3_w, att3_b):
    raise NotImplementedError("write your pallas kernel here")


# speedup vs baseline: 4.8296x; 4.8296x over previous
import functools

import jax
import jax.numpy as jnp
from jax import lax
from jax.experimental import pallas as pl
from jax.experimental.pallas import tpu as pltpu
from jax.experimental.pallas import tpu_sc as plsc

NC = 2
NS = 16
NW = NC * NS
CHUNK = 128


def _sc_gather_body(nchunks, r2e_hbm, idx_hbm, u2e_hbm, nodes_hbm,
                    e_ur_hbm, urep_hbm, idx_v, nidx_v, buf_a, buf_b,
                    ubuf, gsem_a, gsem_b, usem):
  wid = lax.axis_index("s") * NC + lax.axis_index("c")
  rows_per_w = nchunks * CHUNK
  base = pl.multiple_of(wid * rows_per_w, CHUNK)

  pltpu.sync_copy(idx_hbm.at[wid], idx_v)
  pltpu.sync_copy(nodes_hbm.at[wid], nidx_v)

  ucopy = pltpu.async_copy(u2e_hbm.at[nidx_v], ubuf, usem)

  first = pltpu.async_copy(r2e_hbm.at[idx_v.at[0]], buf_a, gsem_a)

  def step(j, _):
    even = (j % 2) == 0

    @pl.when(jnp.logical_and(even, j + 1 < nchunks))
    def _():
      pltpu.async_copy(r2e_hbm.at[idx_v.at[j + 1]], buf_b, gsem_b)

    @pl.when(jnp.logical_and(jnp.logical_not(even), j + 1 < nchunks))
    def _():
      pltpu.async_copy(r2e_hbm.at[idx_v.at[j + 1]], buf_a, gsem_a)

    row0 = pl.multiple_of(base + j * CHUNK, CHUNK)

    @pl.when(even)
    def _():
      pltpu.make_async_copy(r2e_hbm.at[idx_v.at[0]], buf_a, gsem_a).wait()
      pltpu.sync_copy(buf_a, e_ur_hbm.at[pl.ds(row0, CHUNK)])

    @pl.when(jnp.logical_not(even))
    def _():
      pltpu.make_async_copy(r2e_hbm.at[idx_v.at[0]], buf_b, gsem_b).wait()
      pltpu.sync_copy(buf_b, e_ur_hbm.at[pl.ds(row0, CHUNK)])

    return 0

  lax.fori_loop(0, nchunks, step, 0)
  del first

  ucopy.wait()
  ubase = pl.multiple_of(wid * CHUNK, CHUNK)
  pltpu.sync_copy(ubuf, urep_hbm.at[pl.ds(ubase, CHUNK)])


def _sc_gather(r2e_w, idx3, u2e_w, nodes2):
  nw, nchunks, _ = idx3.shape
  bl = nw * nchunks * CHUNK
  b = nodes2.shape[0] * nodes2.shape[1]
  d = r2e_w.shape[1]
  mesh = plsc.VectorSubcoreMesh(core_axis_name="c", subcore_axis_name="s",
                                num_cores=NC, num_subcores=NS)
  f = pl.kernel(
      functools.partial(_sc_gather_body, nchunks),
      out_type=(jax.ShapeDtypeStruct((bl, d), jnp.float32),
                jax.ShapeDtypeStruct((b, d), jnp.float32)),
      mesh=mesh,
      compiler_params=pltpu.CompilerParams(use_tc_tiling_on_sc=False),
      scratch_types=[
          pltpu.VMEM((nchunks, CHUNK), jnp.int32),
          pltpu.VMEM((CHUNK,), jnp.int32),
          pltpu.VMEM((CHUNK, d), jnp.float32),
          pltpu.VMEM((CHUNK, d), jnp.float32),
          pltpu.VMEM((CHUNK, d), jnp.float32),
          pltpu.SemaphoreType.DMA,
          pltpu.SemaphoreType.DMA,
          pltpu.SemaphoreType.DMA,
      ],
  )
  return f(r2e_w, idx3, u2e_w, nodes2)


def _tc_body(L, R, e_ref, rid_ref, u_ref, rat_ref, w1_ref, b1_ref, w2_ref,
             b2_ref, a1w_ref, a1b_ref, a2w_ref, a2b_ref, a3r_ref, out_ref):
  d = e_ref.shape[1]
  rows = e_ref.shape[0]
  bb = rows // L

  e = e_ref[...]
  rid = rid_ref[...]
  w1a = w1_ref[0:d, :]
  w1b = w1_ref[d:2 * d, :]

  m1 = jnp.dot(rat_ref[...], w1b, preferred_element_type=jnp.float32)
  m1 = m1 + b1_ref[...]
  er = jnp.zeros((rows, d), jnp.float32)
  for r in range(R):
    er = er + jnp.where(rid == r, m1[r:r + 1, :], 0.0)

  x = jnp.maximum(
      jnp.dot(e, w1a, preferred_element_type=jnp.float32) + er, 0.0)
  o = jnp.maximum(
      jnp.dot(x, w2_ref[...], preferred_element_type=jnp.float32)
      + b2_ref[...], 0.0)

  u = u_ref[...]
  a1wa = a1w_ref[0:d, :]
  a1wb = a1w_ref[d:2 * d, :]
  uproj = jnp.dot(u, a1wb, preferred_element_type=jnp.float32)

  rgrp = lax.broadcasted_iota(jnp.int32, (rows, bb), 0) // L
  cidx = lax.broadcasted_iota(jnp.int32, (rows, bb), 1)
  P = (rgrp == cidx).astype(jnp.float32)
  rgrp_t = lax.broadcasted_iota(jnp.int32, (bb, rows), 1) // L
  cidx_t = lax.broadcasted_iota(jnp.int32, (bb, rows), 0)
  Q = (rgrp_t == cidx_t).astype(jnp.float32)

  a1 = jnp.maximum(
      jnp.dot(o, a1wa, preferred_element_type=jnp.float32)
      + jnp.dot(P, uproj, preferred_element_type=jnp.float32)
      + a1b_ref[...], 0.0)
  a2 = jnp.maximum(
      jnp.dot(a1, a2w_ref[...], preferred_element_type=jnp.float32)
      + a2b_ref[...], 0.0)

  logit = jnp.sum(a2 * a3r_ref[...], axis=1, keepdims=True)
  gmax = jnp.max(logit)
  ex = jnp.broadcast_to(jnp.exp(logit - gmax), (rows, d))
  den = jnp.dot(P, jnp.dot(Q, ex, preferred_element_type=jnp.float32),
                preferred_element_type=jnp.float32)
  wgt = ex / den

  out_ref[...] = jnp.dot(Q, o * wgt, preferred_element_type=jnp.float32)


def _tc_dense(e_ur, rid, ur_rep, rat_w, w1, b1, w2, b2, a1w, a1b, a2w, a2b,
              a3r, L, BB):
  bl, d = e_ur.shape
  b = ur_rep.shape[0]
  nblk = b // BB
  rows = BB * L
  R = rat_w.shape[0]

  full = lambda shape: pl.BlockSpec(shape, lambda i: (0, 0))
  return pl.pallas_call(
      functools.partial(_tc_body, L, R),
      grid=(nblk,),
      in_specs=[
          pl.BlockSpec((rows, d), lambda i: (i, 0)),
          pl.BlockSpec((rows, 1), lambda i: (i, 0)),
          pl.BlockSpec((BB, d), lambda i: (i, 0)),
          full(rat_w.shape),
          full(w1.shape),
          full(b1.shape),
          full(w2.shape),
          full(b2.shape),
          full(a1w.shape),
          full(a1b.shape),
          full(a2w.shape),
          full(a2b.shape),
          full(a3r.shape),
      ],
      out_specs=pl.BlockSpec((BB, d), lambda i: (i, 0)),
      out_shape=jax.ShapeDtypeStruct((b, d), jnp.float32),
  )(e_ur, rid, ur_rep, rat_w, w1, b1, w2, b2, a1w, a1b, a2w, a2b, a3r)


def kernel(nodes, ur_history_lists, rating_history_lists, u2e_w, r2e_w,
           rating2e_w, w_r1_w, w_r1_b, w_r2_w, w_r2_b, att1_w, att1_b,
           att2_w, att2_b, att3_w, att3_b):
  B, L = ur_history_lists.shape
  D = u2e_w.shape[1]
  BL = B * L
  nchunks = BL // (NW * CHUNK)

  idx3 = ur_history_lists.astype(jnp.int32).reshape(NW, nchunks, CHUNK)
  nodes2 = nodes.astype(jnp.int32).reshape(NW, B // NW)
  e_ur, ur_rep = _sc_gather(r2e_w, idx3, u2e_w, nodes2)

  rid = rating_history_lists.astype(jnp.int32).reshape(BL, 1)
  return _tc_dense(
      e_ur, rid, ur_rep, rating2e_w,
      w_r1_w, w_r1_b.reshape(1, D), w_r2_w, w_r2_b.reshape(1, D),
      att1_w, att1_b.reshape(1, D), att2_w, att2_b.reshape(1, D),
      att3_w.reshape(1, D), L=L, BB=128)
